# speedup vs baseline: 8.6974x; 8.6974x over previous
import functools

import jax
import jax.numpy as jnp
from jax.experimental import pallas as pl
from jax.experimental.pallas import tpu as pltpu

_IN = 128
_STEPS = 3
_G = 512
_B = 512


def _set2set_kernel(x_ref, b_ref, wih_ref, whh_ref, bias_ref, out_ref,
                    m_ref, s_ref, r_ref, h_ref, c_ref, q_ref, *, nb):
    t = pl.program_id(0)
    b = pl.program_id(1)

    @pl.when(b == 0)
    def _lstm():
        first = t == 0
        h_prev = jnp.where(first, 0.0, h_ref[...])
        c_prev = jnp.where(first, 0.0, c_ref[...])
        q_prev = jnp.where(first, 0.0, q_ref[...])
        r_prev = jnp.where(first, 0.0,
                           r_ref[...] / (s_ref[...] + 1e-16))
        q_star = jnp.concatenate([q_prev, r_prev], axis=1)
        gates = (jax.lax.dot_general(
                     q_star, wih_ref[...], (((1,), (1,)), ((), ())),
                     preferred_element_type=jnp.float32) +
                 jax.lax.dot_general(
                     h_prev, whh_ref[...], (((1,), (1,)), ((), ())),
                     preferred_element_type=jnp.float32) +
                 bias_ref[...])
        i_g = jax.nn.sigmoid(gates[:, 0 * _IN:1 * _IN])
        f_g = jax.nn.sigmoid(gates[:, 1 * _IN:2 * _IN])
        g_g = jnp.tanh(gates[:, 2 * _IN:3 * _IN])
        o_g = jax.nn.sigmoid(gates[:, 3 * _IN:4 * _IN])
        c_new = f_g * c_prev + i_g * g_g
        h_new = o_g * jnp.tanh(c_new)
        c_ref[...] = c_new
        h_ref[...] = h_new
        q_ref[...] = h_new
        m_ref[...] = jnp.full((_G, 1), -jnp.inf, jnp.float32)
        s_ref[...] = jnp.zeros((_G, 1), jnp.float32)
        r_ref[...] = jnp.zeros((_G, _IN), jnp.float32)

    xb = x_ref[...]
    bb = b_ref[0]
    q = q_ref[...]
    qxT = jax.lax.dot_general(q, xb, (((1,), (1,)), ((), ())),
                              preferred_element_type=jnp.float32)
    gids = jax.lax.broadcasted_iota(jnp.int32, (_G, _B), 0)
    mask = gids == bb
    e_row = jnp.sum(jnp.where(mask, qxT, 0.0), axis=0, keepdims=True)
    m_old = m_ref[...]
    m_blk = jnp.max(jnp.where(mask, e_row, -jnp.inf), axis=1,
                    keepdims=True)
    m_new = jnp.maximum(m_old, m_blk)
    gath = jnp.sum(jnp.where(mask, m_new, 0.0), axis=0, keepdims=True)
    ex_row = jnp.exp(e_row - gath)
    mex = jnp.where(mask, ex_row, 0.0)
    s_blk = jnp.sum(mex, axis=1, keepdims=True)
    scale = jnp.where(m_old > -jnp.inf, jnp.exp(m_old - m_new), 0.0)
    s_ref[...] = s_ref[...] * scale + s_blk
    r_ref[...] = (r_ref[...] * scale +
                  jax.lax.dot_general(mex, xb, (((1,), (0,)), ((), ())),
                                      preferred_element_type=jnp.float32))
    m_ref[...] = m_new

    @pl.when((t == _STEPS - 1) & (b == nb - 1))
    def _final():
        r_fin = r_ref[...] / (s_ref[...] + 1e-16)
        out_ref[...] = jnp.concatenate([q_ref[...], r_fin], axis=1)


def kernel(x, batch, W_ih, W_hh, b_ih, b_hh):
    n = x.shape[0]
    nb = (n + _B - 1) // _B
    n_pad = nb * _B
    x_p = jnp.pad(x, ((0, n_pad - n), (0, 0)))
    batch_p = jnp.pad(batch, (0, n_pad - n), constant_values=_G)
    batch_p = batch_p.reshape(nb, 1, _B)
    bias = (b_ih + b_hh).reshape(1, 4 * _IN)

    grid = (_STEPS, nb)
    out = pl.pallas_call(
        functools.partial(_set2set_kernel, nb=nb),
        grid=grid,
        in_specs=[
            pl.BlockSpec((_B, _IN), lambda t, b: (b, 0)),
            pl.BlockSpec((1, 1, _B), lambda t, b: (b, 0, 0)),
            pl.BlockSpec((4 * _IN, 2 * _IN), lambda t, b: (0, 0)),
            pl.BlockSpec((4 * _IN, _IN), lambda t, b: (0, 0)),
            pl.BlockSpec((1, 4 * _IN), lambda t, b: (0, 0)),
        ],
        out_specs=pl.BlockSpec((_G, 2 * _IN), lambda t, b: (0, 0)),
        out_shape=jax.ShapeDtypeStruct((_G, 2 * _IN), jnp.float32),
        scratch_shapes=[
            pltpu.VMEM((_G, 1), jnp.float32),
            pltpu.VMEM((_G, 1), jnp.float32),
            pltpu.VMEM((_G, _IN), jnp.float32),
            pltpu.VMEM((_G, _IN), jnp.float32),
            pltpu.VMEM((_G, _IN), jnp.float32),
            pltpu.VMEM((_G, _IN), jnp.float32),
        ],
    )(x_p, batch_p, W_ih, W_hh, bias)
    return out
